# baseline (speedup 1.0000x reference)
import jax
import jax.numpy as jnp
from jax import lax
from jax.experimental import pallas as pl
from jax.experimental.pallas import tpu as pltpu
from jax.experimental.pallas import tpu_sc as plsc

B = 32
C_IN = 3
H = 512
W = 512
CH = 448
CW = 448
NUM_CROPS = 2

NW = 32
TRIPLES = B * NUM_CROPS * C_IN
PER_W = TRIPLES // NW
KR = 32
NCHUNK = CH // KR
NU = PER_W * NCHUNK
LANES = 16
NJ = CW // LANES
WIN = CW + 8
NBI = 4
NBO = 2


def _body(inds_hbm, in_hbm, out_hbm, inds_v,
          ib0, ib1, ib2, ib3, ob0, ob1,
          isem0, isem1, isem2, isem3, osem0, osem1):
    wid = lax.axis_index("s") * 2 + lax.axis_index("c")
    pltpu.sync_copy(inds_hbm, inds_v.at[pl.ds(0, B * NUM_CROPS * 2)])
    lanes = lax.iota(jnp.int32, LANES)
    ibufs, obufs = (ib0, ib1, ib2, ib3), (ob0, ob1)
    isems, osems = (isem0, isem1, isem2, isem3), (osem0, osem1)

    def params(u):
        j = u // NCHUNK
        k = u % NCHUNK
        t = wid * PER_W + j
        b = t // (NUM_CROPS * C_IN)
        r = t % (NUM_CROPS * C_IN)
        n = r // C_IN
        c = r % C_IN
        hw = inds_v[pl.ds((b * NUM_CROPS + n) * 2, LANES)]
        h0 = hw[0]
        w0 = hw[1]
        w8 = pl.multiple_of((w0 // 8) * 8, 8)
        return b, n, c, k, h0, w0 - w8, w8

    def in_copy(u, s):
        b, _, c, k, h0, _, w8 = params(u)
        return pltpu.make_async_copy(
            in_hbm.at[b, c, pl.ds(h0 + k * KR, KR), pl.ds(w8, WIN)],
            ibufs[s], isems[s])

    def out_copy(u, s):
        b, n, c, k, _, _, _ = params(u)
        return pltpu.make_async_copy(
            obufs[s],
            out_hbm.at[b * NUM_CROPS + n, c, pl.ds(k * KR, KR), :],
            osems[s])

    def compute(u, si, so):
        _, _, _, _, _, d, _ = params(u)
        ib, ob = ibufs[si], obufs[so]
        col0 = d + lanes

        @plsc.parallel_loop(0, KR, unroll=4)
        def _(rr):
            row = jnp.full((LANES,), rr, jnp.int32)
            for jj in range(NJ):
                v = plsc.load_gather(ib, [row, col0 + jj * LANES])
                ob[rr, pl.ds(jj * LANES, LANES)] = v

    for w in range(NBI - 1):
        in_copy(w, w).start()

    def grp_body(p, _):
        for s in range(NBI):
            u = NBI * p + s

            @pl.when(u + NBI - 1 < NU)
            def _():
                in_copy(u + NBI - 1, (s + NBI - 1) % NBI).start()

            in_copy(u, s).wait()
            so = s % NBO

            @pl.when(u >= NBO)
            def _():
                out_copy(u - NBO, so).wait()

            compute(u, s, so)
            out_copy(u, so).start()
        return 0

    lax.fori_loop(0, NU // NBI, grp_body, 0)
    out_copy(NU - 2, (NU - 2) % NBO).wait()
    out_copy(NU - 1, (NU - 1) % NBO).wait()


def kernel(inputs, crop_inds):
    mesh = plsc.VectorSubcoreMesh(core_axis_name="c", subcore_axis_name="s",
                                  num_cores=2, num_subcores=16)
    f = pl.kernel(
        _body,
        out_type=jax.ShapeDtypeStruct((B * NUM_CROPS, C_IN, CH, CW),
                                      jnp.float32),
        mesh=mesh,
        compiler_params=pltpu.CompilerParams(use_tc_tiling_on_sc=False,
                                             needs_layout_passes=False),
        scratch_types=(
            [pltpu.VMEM((B * NUM_CROPS * 2 + LANES,), jnp.int32)]
            + [pltpu.VMEM((KR, WIN), jnp.float32)] * NBI
            + [pltpu.VMEM((KR, CW), jnp.float32)] * NBO
            + [pltpu.SemaphoreType.DMA] * (NBI + NBO)
        ),
    )
    return f(crop_inds.reshape(-1).astype(jnp.int32), inputs)
